# baseline (speedup 1.0000x reference)
import functools

import jax
import jax.numpy as jnp
from jax import lax
from jax.experimental import pallas as pl
from jax.experimental.pallas import tpu as pltpu
from jax.experimental.pallas import tpu_sc as plsc

_NC = 2
_NS = 16
_NW = _NC * _NS

_K = 8
_NBUF = 2
_LP = 64


@functools.lru_cache(maxsize=None)
def _build_fused(N, L, D, V):
    RPW = N // _NW
    FPW = RPW * L
    NCHUNK = RPW // _K
    CF = _K * L

    mesh = plsc.VectorSubcoreMesh(core_axis_name="c", subcore_axis_name="s")

    @functools.partial(
        pl.kernel,
        mesh=mesh,
        compiler_params=pltpu.CompilerParams(
            use_tc_tiling_on_sc=False, needs_layout_passes=False),
        out_type=(
            jax.ShapeDtypeStruct((N * L, 2, D), jnp.float32),
        ),
        scratch_types=[
            pltpu.VMEM((RPW,), jnp.int32),
            pltpu.VMEM((RPW, _LP), jnp.int32),
            pltpu.VMEM((FPW,), jnp.int32),
            pltpu.VMEM((FPW,), jnp.int32),
            pltpu.VMEM((16,), jnp.int32),
            pltpu.VMEM((16,), jnp.int32),
            pltpu.VMEM((_NBUF, CF, D), jnp.float32),
            pltpu.VMEM((_NBUF, CF, D), jnp.float32),
            pltpu.SemaphoreType.DMA,
            pltpu.SemaphoreType.DMA((_NBUF,)),
            pltpu.SemaphoreType.DMA((_NBUF,)),
            pltpu.SemaphoreType.DMA((_NBUF,)),
            pltpu.SemaphoreType.DMA((_NBUF,)),
        ],
    )
    def fused(mat_hbm, ct_hbm, emb_hbm, idx_hbm,
              out_hbm,
              idx_v, ct_v, mat_idx, ct_idx, rrow, rt, mbuf, ebuf,
              sem0, msem, esem, wmsem, wesem):
        wid = lax.axis_index("s") * _NC + lax.axis_index("c")
        base = wid * RPW
        fbase = base * L

        pltpu.sync_copy(idx_hbm.at[pl.ds(base, RPW)], idx_v)
        pltpu.async_copy(ct_hbm.at[idx_v], ct_v, sem0).wait()

        rrow[...] = jnp.zeros((16,), jnp.int32)
        rt[...] = lax.iota(jnp.int32, 16)

        def build(k, _):
            row = rrow[...]
            t = rt[...]
            srow = plsc.load_gather(idx_v, [row])
            mat_idx[pl.ds(k * 16, 16)] = srow * L + t
            ct_idx[pl.ds(k * 16, 16)] = plsc.load_gather(ct_v, [row, t])
            t = t + 16
            wrap = (t >= L).astype(jnp.int32)
            rt[...] = t - wrap * L
            rrow[...] = row + wrap
            return 0
        lax.fori_loop(0, FPW // 16, build, 0)

        gathers = [None] * _NBUF
        writes = [None] * _NBUF

        def start(c):
            b = c % _NBUF
            h1 = pltpu.async_copy(
                mat_hbm.at[mat_idx.at[pl.ds(c * CF, CF)]], mbuf.at[b],
                msem.at[b])
            h2 = pltpu.async_copy(
                emb_hbm.at[ct_idx.at[pl.ds(c * CF, CF)]], ebuf.at[b],
                esem.at[b])
            gathers[b] = (h1, h2)

        def retire(c):
            b = c % _NBUF
            h1, h2 = gathers[b]
            o0 = fbase + c * CF
            h1.wait()
            w1 = pltpu.async_copy(
                mbuf.at[b], out_hbm.at[pl.ds(o0, CF), 0, :], wmsem.at[b])
            h2.wait()
            w2 = pltpu.async_copy(
                ebuf.at[b], out_hbm.at[pl.ds(o0, CF), 1, :], wesem.at[b])
            writes[b] = (w1, w2)

        for c in range(NCHUNK):
            b = c % _NBUF
            if c >= _NBUF:
                w1, w2 = writes[b]
                w1.wait()
                w2.wait()
            start(c)
            if c >= 1:
                retire(c - 1)
        retire(NCHUNK - 1)
        for b in range(_NBUF):
            w1, w2 = writes[b]
            w1.wait()
            w2.wait()

    return fused


def kernel(matrix, length, core_terms, emb):
    G, B, L, D = matrix.shape
    N = G * B
    V = emb.shape[0]

    length_flat = length.reshape(-1)
    idx_sort = jnp.argsort(-length_flat).astype(jnp.int32)
    idx_unsort = jnp.argsort(idx_sort).astype(jnp.int32)
    length_sorted = jnp.take(length_flat, idx_sort)

    mat = matrix.reshape(N * L, D).astype(jnp.float32)
    ct = jnp.pad(core_terms.reshape(N, L), ((0, 0), (0, _LP - L)))
    (x4,) = _build_fused(N, L, D, V)(
        mat, ct, emb.astype(jnp.float32), idx_sort)
    return x4.reshape(N, L, 2 * D), length_sorted, idx_unsort


# speedup vs baseline: 1.2415x; 1.2415x over previous
import functools

import jax
import jax.numpy as jnp
from jax import lax
from jax.experimental import pallas as pl
from jax.experimental.pallas import tpu as pltpu
from jax.experimental.pallas import tpu_sc as plsc

_NC = 2
_NS = 16
_NW = _NC * _NS

_NBUF = 2
_LP = 64


@functools.lru_cache(maxsize=None)
def _build_fused(N, L, D, V):
    RPW = N // _NW
    FPW = RPW * L

    mesh = plsc.VectorSubcoreMesh(core_axis_name="c", subcore_axis_name="s")

    @functools.partial(
        pl.kernel,
        mesh=mesh,
        compiler_params=pltpu.CompilerParams(
            use_tc_tiling_on_sc=False, needs_layout_passes=False),
        out_type=(
            jax.ShapeDtypeStruct((L, N, 2 * D), jnp.float32),
        ),
        scratch_types=[
            pltpu.VMEM((RPW,), jnp.int32),
            pltpu.VMEM((RPW, _LP), jnp.int32),
            pltpu.VMEM((FPW,), jnp.int32),
            pltpu.VMEM((FPW,), jnp.int32),
            pltpu.VMEM((_NBUF, RPW, D), jnp.float32),
            pltpu.VMEM((_NBUF, RPW, D), jnp.float32),
            pltpu.SemaphoreType.DMA,
            pltpu.SemaphoreType.DMA((_NBUF,)),
            pltpu.SemaphoreType.DMA((_NBUF,)),
            pltpu.SemaphoreType.DMA((_NBUF,)),
            pltpu.SemaphoreType.DMA((_NBUF,)),
        ],
    )
    def fused(mat_hbm, ct_hbm, emb_hbm, idx_hbm,
              out_hbm,
              idx_v, ct_v, mat_idx, ct_idx, mbuf, ebuf,
              sem0, msem, esem, wmsem, wesem):
        wid = lax.axis_index("s") * _NC + lax.axis_index("c")
        base = wid * RPW

        pltpu.sync_copy(idx_hbm.at[pl.ds(base, RPW)], idx_v)
        pltpu.async_copy(ct_hbm.at[idx_v], ct_v, sem0).wait()

        def build(k, _):
            q = k * 16 + lax.iota(jnp.int32, 16)
            t = q >> 7
            j = q & (RPW - 1)
            srow = plsc.load_gather(idx_v, [j])
            mat_idx[pl.ds(k * 16, 16)] = srow * L + t
            ct_idx[pl.ds(k * 16, 16)] = plsc.load_gather(ct_v, [j, t])
            return 0
        lax.fori_loop(0, FPW // 16, build, 0)

        gathers = [None] * _NBUF
        writes = [None] * _NBUF

        def start(t):
            b = t % _NBUF
            h1 = pltpu.async_copy(
                mat_hbm.at[mat_idx.at[pl.ds(t * RPW, RPW)]], mbuf.at[b],
                msem.at[b])
            h2 = pltpu.async_copy(
                emb_hbm.at[ct_idx.at[pl.ds(t * RPW, RPW)]], ebuf.at[b],
                esem.at[b])
            gathers[b] = (h1, h2)

        def retire(t):
            b = t % _NBUF
            h1, h2 = gathers[b]
            h1.wait()
            w1 = pltpu.async_copy(
                mbuf.at[b], out_hbm.at[t, pl.ds(base, RPW), pl.ds(0, D)],
                wmsem.at[b])
            h2.wait()
            w2 = pltpu.async_copy(
                ebuf.at[b], out_hbm.at[t, pl.ds(base, RPW), pl.ds(D, D)],
                wesem.at[b])
            writes[b] = (w1, w2)

        for t in range(L):
            b = t % _NBUF
            if t >= _NBUF:
                w1, w2 = writes[b]
                w1.wait()
                w2.wait()
            start(t)
            if t >= 1:
                retire(t - 1)
        retire(L - 1)
        for b in range(_NBUF):
            w1, w2 = writes[b]
            w1.wait()
            w2.wait()

    return fused


def kernel(matrix, length, core_terms, emb):
    G, B, L, D = matrix.shape
    N = G * B
    V = emb.shape[0]

    length_flat = length.reshape(-1)
    idx_sort = jnp.argsort(-length_flat).astype(jnp.int32)
    idx_unsort = jnp.argsort(idx_sort).astype(jnp.int32)
    length_sorted = jnp.take(length_flat, idx_sort)

    mat_flat = lax.optimization_barrier(
        matrix.astype(jnp.float32).reshape(-1))
    emb_flat = lax.optimization_barrier(emb.astype(jnp.float32).reshape(-1))
    mat = mat_flat.reshape(N * L, D)
    emb2 = emb_flat.reshape(V, D)
    ct = jnp.pad(core_terms.reshape(N, L), ((0, 0), (0, _LP - L)))
    (xt,) = _build_fused(N, L, D, V)(mat, ct, emb2, idx_sort)
    x = jnp.transpose(xt, (1, 0, 2))
    return x, length_sorted, idx_unsort
